# baseline (speedup 1.0000x reference)
import functools

import jax
import jax.numpy as jnp
from jax.experimental import pallas as pl
from jax.experimental.pallas import tpu as pltpu

_B, _H, _W, _C = 8, 512, 512, 32
_NP = 256
_CROP = 16
_NCLS = 3
_CC = 64
_EPS = 1e-8
_FLAT = (_CROP // 4) * (_CROP // 4) * 2 * _CC


def _frame_stats(lab, sz, wt):
    lab = lab.reshape(-1)
    rows = (jnp.arange(_H * _W) // _W).astype(jnp.int32)
    cols = (jnp.arange(_H * _W) % _W).astype(jnp.int32)
    y0 = jax.ops.segment_min(rows, lab, _NP)
    x0 = jax.ops.segment_min(cols, lab, _NP)
    y1 = jax.ops.segment_max(rows, lab, _NP) + 1
    x1 = jax.ops.segment_max(cols, lab, _NP) + 1
    bbox = jnp.stack([y0, x0, y1, x1], axis=-1).astype(jnp.float32)
    bbox = bbox / jnp.array([_H, _W, _H, _W], jnp.float32)
    wf = wt.reshape(-1, 1)
    num = jax.ops.segment_sum(sz.reshape(-1, 2) * wf, lab, _NP)
    den = jax.ops.segment_sum(jnp.broadcast_to(wf, (_H * _W, 2)), lab, _NP)
    sizes = (num / (den + _EPS)) / jnp.array([_H, _W], jnp.float32)
    return bbox, sizes


def _crop_frame(feat, boxes):
    lin = (jnp.arange(_CROP) / (_CROP - 1)).astype(jnp.float32)
    py = (boxes[:, 0:1] + (boxes[:, 2:3] - boxes[:, 0:1]) * lin) * (_H - 1)
    px = (boxes[:, 1:2] + (boxes[:, 3:4] - boxes[:, 1:2]) * lin) * (_W - 1)
    py = jnp.clip(py, 0.0, _H - 1.0)
    px = jnp.clip(px, 0.0, _W - 1.0)
    y0i = jnp.clip(jnp.floor(py).astype(jnp.int32), 0, _H - 2)
    x0i = jnp.clip(jnp.floor(px).astype(jnp.int32), 0, _W - 2)
    fy = (py - y0i)[:, :, None, None]
    fx = (px - x0i)[:, None, :, None]
    g = lambda yi, xi: feat[yi[:, :, None], xi[:, None, :]]
    v00, v01 = g(y0i, x0i), g(y0i, x0i + 1)
    v10, v11 = g(y0i + 1, x0i), g(y0i + 1, x0i + 1)
    return (v00 * (1 - fy) * (1 - fx) + v01 * (1 - fy) * fx
            + v10 * fy * (1 - fx) + v11 * fy * fx)


def _conv(x, w, b, stride):
    y = jax.lax.conv_general_dilated(x, w, (stride, stride), 'SAME',
                                     dimension_numbers=('NHWC', 'HWIO', 'NHWC'))
    return jax.nn.relu(y + b)


def _head_body(x_ref, bf_ref, fc1w_ref, fc1b_ref, fc2w_ref, fc2b_ref,
               regw_ref, regb_ref, scow_ref, selr_ref, sels_ref, out_ref):
    x = x_ref[0]
    h = jnp.maximum(jnp.dot(x, fc1w_ref[...],
                            preferred_element_type=jnp.float32)
                    + fc1b_ref[...], 0.0)
    h = jnp.maximum(jnp.dot(h, fc2w_ref[...],
                            preferred_element_type=jnp.float32)
                    + fc2b_ref[...], 0.0)
    reg = jnp.dot(h, regw_ref[...], preferred_element_type=jnp.float32) \
        + regb_ref[...]
    sco = jnp.dot(h, scow_ref[...], preferred_element_type=jnp.float32)
    reg_out = jnp.dot(reg, selr_ref[0], preferred_element_type=jnp.float32)
    sco_out = jnp.dot(sco, sels_ref[0], preferred_element_type=jnp.float32)
    bf = bf_ref[0]
    bh = bf[:, 2:3] - bf[:, 0:1]
    bw = bf[:, 3:4] - bf[:, 1:2]
    bcy = (bf[:, 0:1] + bf[:, 2:3]) * 0.5
    bcx = (bf[:, 1:2] + bf[:, 3:4]) * 0.5
    ncy = bcy + reg_out[:, 0:1] * bh
    ncx = bcx + reg_out[:, 1:2] * bw
    nh = bh * jnp.exp(reg_out[:, 2:3])
    nw = bw * jnp.exp(reg_out[:, 3:4])
    out = jnp.concatenate([
        reg_out[:, 0:4],
        sco_out[:, 0:1],
        ncy - nh * 0.5, ncx - nw * 0.5, ncy + nh * 0.5, ncx + nw * 0.5,
        jnp.zeros((_NP, 7), jnp.float32)], axis=-1)
    out_ref[0] = out


def _head(x, new_bboxes, cls, fc1_w, fc1_b, fc2_w, fc2_b,
          reg_w, reg_b, score_w, score_b):
    regw = jnp.pad(reg_w, ((0, 0), (0, 4)))
    regb = jnp.pad(reg_b, ((0, 4)))
    scow = jnp.pad(score_w + score_b[None, :], ((0, 0), (0, 5)))
    r_idx = jnp.arange(16, dtype=jnp.int32)[None, :, None]
    k_idx = jnp.arange(8, dtype=jnp.int32)[None, None, :]
    tgt = cls[:, None, None] * 4 + k_idx
    selr = ((r_idx == tgt) & (k_idx < 4)).astype(jnp.float32)
    sels = ((jnp.arange(8, dtype=jnp.int32)[None, :, None] == cls[:, None, None])
            & (k_idx == 0)).astype(jnp.float32)
    x3 = x.reshape(_B, _NP, _FLAT)
    full = lambda shape: pl.BlockSpec(shape, lambda i: (0,) * len(shape))
    out = pl.pallas_call(
        _head_body,
        grid=(_B,),
        in_specs=[
            pl.BlockSpec((1, _NP, _FLAT), lambda i: (i, 0, 0)),
            pl.BlockSpec((1, _NP, 4), lambda i: (i, 0, 0)),
            full((_FLAT, 512)), full((512,)),
            full((512, 512)), full((512,)),
            full((512, 16)), full((16,)),
            full((512, 8)),
            pl.BlockSpec((1, 16, 8), lambda i: (i, 0, 0)),
            pl.BlockSpec((1, 8, 8), lambda i: (i, 0, 0)),
        ],
        out_specs=pl.BlockSpec((1, _NP, 16), lambda i: (i, 0, 0)),
        out_shape=jax.ShapeDtypeStruct((_B, _NP, 16), jnp.float32),
        compiler_params=pltpu.CompilerParams(
            dimension_semantics=("parallel",)),
    )(x3, new_bboxes, fc1_w, fc1_b, fc2_w, fc2_b, regw, regb, scow, selr, sels)
    out = out.reshape(_B * _NP, 16)
    return out[:, 0:4], out[:, 4], out[:, 5:9]


def kernel(features, labels_map, sizes_map, weights, cls,
           conv1_w, conv1_b, conv2_w, conv2_b,
           fc1_w, fc1_b, fc2_w, fc2_b, reg_w, reg_b, score_w, score_b):
    bboxes, sizes = jax.vmap(_frame_stats)(labels_map, sizes_map, weights)
    cy = (bboxes[..., 0] + bboxes[..., 2]) * 0.5
    cx = (bboxes[..., 1] + bboxes[..., 3]) * 0.5
    sh, sw = sizes[..., 0], sizes[..., 1]
    new_bboxes = jnp.stack(
        [cy - sh * 0.5, cx - sw * 0.5, cy + sh * 0.5, cx + sw * 0.5], axis=-1)
    crops = jax.vmap(_crop_frame)(features, new_bboxes)
    x = crops.reshape(_B * _NP, _CROP, _CROP, _C)
    x = _conv(x, conv1_w, conv1_b, 2)
    x = _conv(x, conv2_w, conv2_b, 2)
    x = x.reshape(_B * _NP, -1)
    regression_out, regression_scores, regression_bboxes = _head(
        x, new_bboxes, cls, fc1_w, fc1_b, fc2_w, fc2_b,
        reg_w, reg_b, score_w, score_b)
    return new_bboxes, regression_out, regression_scores, regression_bboxes


# speedup vs baseline: 2.9892x; 2.9892x over previous
import functools

import jax
import jax.numpy as jnp
from jax.experimental import pallas as pl
from jax.experimental.pallas import tpu as pltpu

_B, _H, _W, _C = 8, 512, 512, 32
_NP = 256
_CROP = 16
_NCLS = 3
_CC = 64
_EPS = 1e-8
_FLAT = (_CROP // 4) * (_CROP // 4) * 2 * _CC


_BIG = float(2 ** 31)
_RG = 16
_NRG = _H // _RG


def _seg_body(lab_ref, szw_ref, w_ref, out_ref, srow_ref, ccol_ref):
    iota_l = jax.lax.broadcasted_iota(jnp.int32, (_NP, 1), 0)
    ones_row = jnp.ones((1, _W), jnp.float32)
    zeros4 = jnp.zeros((4, _W), jnp.float32)
    ccol_ref[...] = jnp.zeros((_NP, _W), jnp.float32)

    def rg_body(rg, carry):
        for k in range(_RG):
            r = rg * _RG + k
            lab_row = lab_ref[0, pl.ds(r, 1), :]
            eqf = (lab_row == iota_l).astype(jnp.float32)
            ccol_ref[...] += eqf
            wrow = w_ref[0, pl.ds(r, 1), :]
            syrow = szw_ref[0, 0, pl.ds(r, 1), :]
            sxrow = szw_ref[0, 1, pl.ds(r, 1), :]
            vals = jnp.concatenate(
                [syrow * wrow, sxrow * wrow, wrow, ones_row, zeros4], axis=0)
            out = jax.lax.dot_general(
                eqf, vals, (((1,), (1,)), ((), ())),
                precision=jax.lax.Precision.HIGHEST,
                preferred_element_type=jnp.float32)
            srow_ref[pl.ds(rg, 1), :, 8 * k:8 * k + 8] = out[None]
        return carry

    jax.lax.fori_loop(0, _NRG, rg_body, 0)

    tot = jnp.zeros((_NP, 8 * _RG), jnp.float32)
    for rg in range(_NRG):
        tot = tot + srow_ref[rg]
    tot8 = tot.reshape(_NP, _RG, 8).sum(axis=1)
    num_y = tot8[:, 0:1]
    num_x = tot8[:, 1:2]
    den = tot8[:, 2:3]

    lane = jax.lax.broadcasted_iota(jnp.int32, (_NP, 8 * _RG), 1)
    iscnt = (lane % 8) == 3
    ridx_k = (lane // 8).astype(jnp.float32)
    y0m = jnp.full((_NP, 8 * _RG), _BIG, jnp.float32)
    y1m = jnp.full((_NP, 8 * _RG), -_BIG, jnp.float32)
    for rg in range(_NRG):
        chunk = srow_ref[rg]
        pres = iscnt & (chunk > 0)
        ridx = ridx_k + float(rg * _RG)
        y0m = jnp.minimum(y0m, jnp.where(pres, ridx, _BIG))
        y1m = jnp.maximum(y1m, jnp.where(pres, ridx, -_BIG))
    y0 = jnp.min(y0m, axis=1, keepdims=True)
    y1 = jnp.max(y1m, axis=1, keepdims=True)

    cc = ccol_ref[...]
    cidx = jax.lax.broadcasted_iota(jnp.int32, (_NP, _W), 1).astype(jnp.float32)
    x0 = jnp.min(jnp.where(cc > 0, cidx, _BIG), axis=1, keepdims=True)
    x1 = jnp.max(jnp.where(cc > 0, cidx, -_BIG), axis=1, keepdims=True)

    inv = 1.0 / _H
    b0 = y0 * inv
    b1 = x0 * inv
    b2 = (y1 + 1.0) * inv
    b3 = (x1 + 1.0) * inv
    sy = num_y / (den + _EPS) * inv
    sx = num_x / (den + _EPS) * inv
    cy = (b0 + b2) * 0.5
    cx = (b1 + b3) * 0.5
    nb = jnp.concatenate(
        [cy - sy * 0.5, cx - sx * 0.5, cy + sy * 0.5, cx + sx * 0.5], axis=1)
    out_ref[0] = nb


def _seg_stats(labels_map, sizes_map, weights):
    szw = sizes_map.transpose(0, 3, 1, 2)
    wt = weights.reshape(_B, _H, _W)
    return pl.pallas_call(
        _seg_body,
        grid=(_B,),
        in_specs=[
            pl.BlockSpec((1, _H, _W), lambda i: (i, 0, 0)),
            pl.BlockSpec((1, 2, _H, _W), lambda i: (i, 0, 0, 0)),
            pl.BlockSpec((1, _H, _W), lambda i: (i, 0, 0)),
        ],
        out_specs=pl.BlockSpec((1, _NP, 4), lambda i: (i, 0, 0)),
        out_shape=jax.ShapeDtypeStruct((_B, _NP, 4), jnp.float32),
        scratch_shapes=[
            pltpu.VMEM((_NRG, _NP, 8 * _RG), jnp.float32),
            pltpu.VMEM((_NP, _W), jnp.float32),
        ],
        compiler_params=pltpu.CompilerParams(
            dimension_semantics=("parallel",)),
    )(labels_map, szw, wt)


def _crop_frame(feat, boxes):
    lin = (jnp.arange(_CROP) / (_CROP - 1)).astype(jnp.float32)
    py = (boxes[:, 0:1] + (boxes[:, 2:3] - boxes[:, 0:1]) * lin) * (_H - 1)
    px = (boxes[:, 1:2] + (boxes[:, 3:4] - boxes[:, 1:2]) * lin) * (_W - 1)
    py = jnp.clip(py, 0.0, _H - 1.0)
    px = jnp.clip(px, 0.0, _W - 1.0)
    y0i = jnp.clip(jnp.floor(py).astype(jnp.int32), 0, _H - 2)
    x0i = jnp.clip(jnp.floor(px).astype(jnp.int32), 0, _W - 2)
    fy = (py - y0i)[:, :, None, None]
    fx = (px - x0i)[:, None, :, None]
    g = lambda yi, xi: feat[yi[:, :, None], xi[:, None, :]]
    v00, v01 = g(y0i, x0i), g(y0i, x0i + 1)
    v10, v11 = g(y0i + 1, x0i), g(y0i + 1, x0i + 1)
    return (v00 * (1 - fy) * (1 - fx) + v01 * (1 - fy) * fx
            + v10 * fy * (1 - fx) + v11 * fy * fx)


def _conv(x, w, b, stride):
    y = jax.lax.conv_general_dilated(x, w, (stride, stride), 'SAME',
                                     dimension_numbers=('NHWC', 'HWIO', 'NHWC'))
    return jax.nn.relu(y + b)


def _head_body(x_ref, bf_ref, fc1w_ref, fc1b_ref, fc2w_ref, fc2b_ref,
               regw_ref, regb_ref, scow_ref, selr_ref, sels_ref, out_ref):
    x = x_ref[0]
    h = jnp.maximum(jnp.dot(x, fc1w_ref[...],
                            preferred_element_type=jnp.float32)
                    + fc1b_ref[...], 0.0)
    h = jnp.maximum(jnp.dot(h, fc2w_ref[...],
                            preferred_element_type=jnp.float32)
                    + fc2b_ref[...], 0.0)
    reg = jnp.dot(h, regw_ref[...], preferred_element_type=jnp.float32) \
        + regb_ref[...]
    sco = jnp.dot(h, scow_ref[...], preferred_element_type=jnp.float32)
    reg_out = jnp.dot(reg, selr_ref[0], preferred_element_type=jnp.float32)
    sco_out = jnp.dot(sco, sels_ref[0], preferred_element_type=jnp.float32)
    bf = bf_ref[0]
    bh = bf[:, 2:3] - bf[:, 0:1]
    bw = bf[:, 3:4] - bf[:, 1:2]
    bcy = (bf[:, 0:1] + bf[:, 2:3]) * 0.5
    bcx = (bf[:, 1:2] + bf[:, 3:4]) * 0.5
    ncy = bcy + reg_out[:, 0:1] * bh
    ncx = bcx + reg_out[:, 1:2] * bw
    nh = bh * jnp.exp(reg_out[:, 2:3])
    nw = bw * jnp.exp(reg_out[:, 3:4])
    out = jnp.concatenate([
        reg_out[:, 0:4],
        sco_out[:, 0:1],
        ncy - nh * 0.5, ncx - nw * 0.5, ncy + nh * 0.5, ncx + nw * 0.5,
        jnp.zeros((_NP, 7), jnp.float32)], axis=-1)
    out_ref[0] = out


def _head(x, new_bboxes, cls, fc1_w, fc1_b, fc2_w, fc2_b,
          reg_w, reg_b, score_w, score_b):
    regw = jnp.pad(reg_w, ((0, 0), (0, 4)))
    regb = jnp.pad(reg_b, ((0, 4)))
    scow = jnp.pad(score_w + score_b[None, :], ((0, 0), (0, 5)))
    r_idx = jnp.arange(16, dtype=jnp.int32)[None, :, None]
    k_idx = jnp.arange(8, dtype=jnp.int32)[None, None, :]
    tgt = cls[:, None, None] * 4 + k_idx
    selr = ((r_idx == tgt) & (k_idx < 4)).astype(jnp.float32)
    sels = ((jnp.arange(8, dtype=jnp.int32)[None, :, None] == cls[:, None, None])
            & (k_idx == 0)).astype(jnp.float32)
    x3 = x.reshape(_B, _NP, _FLAT)
    full = lambda shape: pl.BlockSpec(shape, lambda i: (0,) * len(shape))
    out = pl.pallas_call(
        _head_body,
        grid=(_B,),
        in_specs=[
            pl.BlockSpec((1, _NP, _FLAT), lambda i: (i, 0, 0)),
            pl.BlockSpec((1, _NP, 4), lambda i: (i, 0, 0)),
            full((_FLAT, 512)), full((512,)),
            full((512, 512)), full((512,)),
            full((512, 16)), full((16,)),
            full((512, 8)),
            pl.BlockSpec((1, 16, 8), lambda i: (i, 0, 0)),
            pl.BlockSpec((1, 8, 8), lambda i: (i, 0, 0)),
        ],
        out_specs=pl.BlockSpec((1, _NP, 16), lambda i: (i, 0, 0)),
        out_shape=jax.ShapeDtypeStruct((_B, _NP, 16), jnp.float32),
        compiler_params=pltpu.CompilerParams(
            dimension_semantics=("parallel",)),
    )(x3, new_bboxes, fc1_w, fc1_b, fc2_w, fc2_b, regw, regb, scow, selr, sels)
    out = out.reshape(_B * _NP, 16)
    return out[:, 0:4], out[:, 4], out[:, 5:9]


def kernel(features, labels_map, sizes_map, weights, cls,
           conv1_w, conv1_b, conv2_w, conv2_b,
           fc1_w, fc1_b, fc2_w, fc2_b, reg_w, reg_b, score_w, score_b):
    new_bboxes = _seg_stats(labels_map, sizes_map, weights)
    crops = jax.vmap(_crop_frame)(features, new_bboxes)
    x = crops.reshape(_B * _NP, _CROP, _CROP, _C)
    x = _conv(x, conv1_w, conv1_b, 2)
    x = _conv(x, conv2_w, conv2_b, 2)
    x = x.reshape(_B * _NP, -1)
    regression_out, regression_scores, regression_bboxes = _head(
        x, new_bboxes, cls, fc1_w, fc1_b, fc2_w, fc2_b,
        reg_w, reg_b, score_w, score_b)
    return new_bboxes, regression_out, regression_scores, regression_bboxes
